# baseline (speedup 1.0000x reference)
import functools

import jax
import jax.numpy as jnp
from jax import lax
from jax.experimental import pallas as pl
from jax.experimental.pallas import tpu as pltpu
from jax.experimental.pallas import tpu_sc as plsc

_F = 64
_FP = 128
_G = 64
_W = 32
_S = 128


def _mesh():
    return plsc.VectorSubcoreMesh(core_axis_name="c", subcore_axis_name="s",
                                  num_cores=2, num_subcores=16)


def _count_body(idx_hbm, out_hbm, idxv, onesv, zv, cnt_sh, *, steps, cntp):
    c = lax.axis_index("c")
    s = lax.axis_index("s")
    wid = c * 16 + s
    pltpu.sync_copy(idx_hbm.at[wid], idxv)
    for k in range(8):
        onesv[pl.ds(16 * k, 16)] = jnp.ones((16,), jnp.int32)

    nz = zv.shape[0]

    @pl.when(s == 0)
    def _():
        def zstep(i, _):
            zv[pl.ds(i * 16, 16)] = jnp.zeros((16,), jnp.int32)
            return 0
        lax.fori_loop(0, nz // 16, zstep, 0)
        for q in range(cntp // nz):
            pltpu.sync_copy(zv, cnt_sh.at[pl.ds(q * nz, nz)])

    plsc.subcore_barrier()

    def step(j, _):
        pltpu.sync_copy(onesv, cnt_sh.at[idxv.at[j]], add=True)
        return 0
    lax.fori_loop(0, steps, step, 0)

    plsc.subcore_barrier()
    sl = cntp // 16
    pltpu.sync_copy(cnt_sh.at[pl.ds(s * sl, sl)],
                    out_hbm.at[pl.ds(c * cntp + s * sl, sl)])


def _sc_count(idx3, cntp):
    steps = idx3.shape[1]
    out = pl.kernel(
        functools.partial(_count_body, steps=steps, cntp=cntp),
        out_type=jax.ShapeDtypeStruct((2 * cntp,), jnp.int32),
        mesh=_mesh(),
        scratch_types=[
            pltpu.VMEM((steps, _S), jnp.int32),
            pltpu.VMEM((_S,), jnp.int32),
            pltpu.VMEM((12800,), jnp.int32),
            pltpu.VMEM_SHARED((cntp,), jnp.int32),
        ],
    )(idx3)
    out = out.reshape(2, cntp)
    return out[0] + out[1]


def _gather2_body(taba, tabb, idxa, idxb, outa, outb,
                  iva, ivb, ra, rb, sa, sb, *, steps):
    c = lax.axis_index("c")
    s = lax.axis_index("s")
    wid = c * 16 + s
    pltpu.sync_copy(idxa.at[wid], iva)
    pltpu.sync_copy(idxb.at[wid], ivb)

    def step(j, _):
        ca = pltpu.async_copy(taba.at[iva.at[j]], ra, sa)
        cb = pltpu.async_copy(tabb.at[ivb.at[j]], rb, sb)
        ca.wait()
        cb.wait()
        pltpu.sync_copy(ra, outa.at[wid, j])
        pltpu.sync_copy(rb, outb.at[wid, j])
        return 0
    lax.fori_loop(0, steps, step, 0)


def _sc_gather2(taba, tabb, idxa3, idxb3):
    steps = idxa3.shape[1]
    sh = jax.ShapeDtypeStruct((_W, steps, _S, _FP), jnp.float32)
    outa, outb = pl.kernel(
        functools.partial(_gather2_body, steps=steps),
        out_type=(sh, sh),
        mesh=_mesh(),
        scratch_types=[
            pltpu.VMEM((steps, _S), jnp.int32),
            pltpu.VMEM((steps, _S), jnp.int32),
            pltpu.VMEM((_S, _FP), jnp.float32),
            pltpu.VMEM((_S, _FP), jnp.float32),
            pltpu.SemaphoreType.DMA,
            pltpu.SemaphoreType.DMA,
        ],
    )(taba, tabb, idxa3, idxb3)
    return (outa.reshape(_W * steps * _S, _FP),
            outb.reshape(_W * steps * _S, _FP))


def _gatherE2_body(taba, tabb, idxa, idxb, outa, outb,
                   iva, ivb, ra, rb, sa, sb, *, steps):
    c = lax.axis_index("c")
    s = lax.axis_index("s")
    wid = c * 16 + s
    pltpu.sync_copy(idxa.at[wid], iva)
    pltpu.sync_copy(idxb.at[wid], ivb)

    def step(j, _):
        ca = pltpu.async_copy(taba.at[iva.at[j]], ra, sa)
        cb = pltpu.async_copy(tabb.at[ivb.at[j]], rb, sb)
        ca.wait()
        cb.wait()
        pltpu.sync_copy(ra, outa.at[wid, j])
        pltpu.sync_copy(rb, outb.at[wid, j])
        return 0
    lax.fori_loop(0, steps, step, 0)


def _sc_gather_e2(tab, idxa3, idxb3):
    steps = idxa3.shape[1]
    sh = jax.ShapeDtypeStruct((_W, steps, _S), jnp.int32)
    outa, outb = pl.kernel(
        functools.partial(_gatherE2_body, steps=steps),
        out_type=(sh, sh),
        mesh=_mesh(),
        scratch_types=[
            pltpu.VMEM((steps, _S), jnp.int32),
            pltpu.VMEM((steps, _S), jnp.int32),
            pltpu.VMEM((_S,), jnp.int32),
            pltpu.VMEM((_S,), jnp.int32),
            pltpu.SemaphoreType.DMA,
            pltpu.SemaphoreType.DMA,
        ],
    )(tab, tab, idxa3, idxb3)
    return outa.reshape(-1), outb.reshape(-1)


def _gather1_body(tab, idx, out, iv, r0, r1, s0, s1, *, steps):
    c = lax.axis_index("c")
    s = lax.axis_index("s")
    wid = c * 16 + s
    pltpu.sync_copy(idx.at[wid], iv)

    def pair(jo, _):
        j0 = 2 * jo
        c0 = pltpu.async_copy(tab.at[iv.at[j0]], r0, s0)
        c1 = pltpu.async_copy(tab.at[iv.at[j0 + 1]], r1, s1)
        c0.wait()
        pltpu.sync_copy(r0, out.at[wid, j0])
        c1.wait()
        pltpu.sync_copy(r1, out.at[wid, j0 + 1])
        return 0
    lax.fori_loop(0, steps // 2, pair, 0)


def _sc_gather1(tab, idx3):
    steps = idx3.shape[1]
    out = pl.kernel(
        functools.partial(_gather1_body, steps=steps),
        out_type=jax.ShapeDtypeStruct((_W, steps, _S, _FP), jnp.float32),
        mesh=_mesh(),
        scratch_types=[
            pltpu.VMEM((steps, _S), jnp.int32),
            pltpu.VMEM((_S, _FP), jnp.float32),
            pltpu.VMEM((_S, _FP), jnp.float32),
            pltpu.SemaphoreType.DMA,
            pltpu.SemaphoreType.DMA,
        ],
    )(tab, idx3)
    return out.reshape(_W * steps * _S, _FP)


def _bprep_body(pos_ref, wp_ref, b_ref):
    b_ref[...] = jnp.dot(pos_ref[...], wp_ref[...],
                         preferred_element_type=jnp.float32)


def _aprep_body(h_ref, b_ref, wh_ref, b1_ref, a_ref):
    a_ref[...] = (jnp.dot(h_ref[...], wh_ref[...],
                          preferred_element_type=jnp.float32)
                  + b_ref[...] + b1_ref[...])


def _bprep(pos, wp, nk):
    n = pos.shape[0]
    return pl.pallas_call(
        _bprep_body,
        grid=(n // nk,),
        in_specs=[pl.BlockSpec((nk, 3), lambda i: (i, 0)),
                  pl.BlockSpec((3, _FP), lambda i: (0, 0))],
        out_specs=pl.BlockSpec((nk, _FP), lambda i: (i, 0)),
        out_shape=jax.ShapeDtypeStruct((n, _FP), jnp.float32),
    )(pos, wp)


def _aprep(h, b, wh, b1, nk):
    n, hw = h.shape
    return pl.pallas_call(
        _aprep_body,
        grid=(n // nk,),
        in_specs=[pl.BlockSpec((nk, hw), lambda i: (i, 0)),
                  pl.BlockSpec((nk, _FP), lambda i: (i, 0)),
                  pl.BlockSpec((hw, _FP), lambda i: (0, 0)),
                  pl.BlockSpec((1, _FP), lambda i: (0, 0))],
        out_specs=pl.BlockSpec((nk, _FP), lambda i: (i, 0)),
        out_shape=jax.ShapeDtypeStruct((n, _FP), jnp.float32),
    )(h, b, wh, b1.reshape(1, _FP))


def _edge_body(dst_ref, xa_ref, xb_ref, w2_ref, cdi_ref, cmi_ref,
               out_ref, cdo_ref, cmo_ref, cdst_ref, cmax_ref, *, ek, nc):
    pid = pl.program_id(0)
    x = jnp.maximum(xa_ref[:, :_F] - xb_ref[:, :_F], 0.0)
    m = jnp.dot(x, w2_ref[...], preferred_element_type=jnp.float32)
    d = dst_ref[...]
    rows = lax.broadcasted_iota(jnp.int32, (ek, 1), 0)
    s = 1
    while s < ek:
        d_s = jnp.where(rows >= s, pltpu.roll(d, s, 0), -1)
        m_s = pltpu.roll(m, s, 0)
        take = d_s == d
        m = jnp.where(take, jnp.maximum(m, m_s), m)
        s *= 2

    @pl.when(pid == 0)
    def _():
        cdst_ref[...] = cdi_ref[...]
        cmax_ref[...] = cmi_ref[...]

    m = jnp.where(d == cdst_ref[...], jnp.maximum(m, cmax_ref[0:1, :]), m)
    out_ref[...] = jnp.concatenate(
        [m, jnp.zeros((ek, _FP - _F), jnp.float32)], axis=1)
    cdst_ref[...] = d[ek - 1:ek, 0:1]
    cmax_ref[0:1, :] = m[ek - 1:ek, :]

    @pl.when(pid == nc - 1)
    def _():
        cdo_ref[...] = cdst_ref[...]
        cmo_ref[...] = cmax_ref[...]


def _edge_scan(dst2, xa, xb, w2, cdi, cmi, ek, half, nedges):
    nc = nedges // ek
    off = half * nc
    return pl.pallas_call(
        functools.partial(_edge_body, ek=ek, nc=nc),
        grid=(nc,),
        in_specs=[pl.BlockSpec((ek, 1), lambda i: (i + off, 0)),
                  pl.BlockSpec((ek, _FP), lambda i: (i, 0)),
                  pl.BlockSpec((ek, _FP), lambda i: (i + off, 0)),
                  pl.BlockSpec((_F, _F), lambda i: (0, 0)),
                  pl.BlockSpec((1, 1), lambda i: (0, 0)),
                  pl.BlockSpec((8, _F), lambda i: (0, 0))],
        out_specs=(pl.BlockSpec((ek, _FP), lambda i: (i, 0)),
                   pl.BlockSpec((1, 1), lambda i: (0, 0)),
                   pl.BlockSpec((8, _F), lambda i: (0, 0))),
        out_shape=(jax.ShapeDtypeStruct((nedges, _FP), jnp.float32),
                   jax.ShapeDtypeStruct((1, 1), jnp.int32),
                   jax.ShapeDtypeStruct((8, _F), jnp.float32)),
        scratch_shapes=[pltpu.VMEM((1, 1), jnp.int32),
                        pltpu.VMEM((8, _F), jnp.float32)],
    )(dst2, xa, xb, w2, cdi, cmi)


def _pool_body(h_ref, bat_ref, r1w_ref, r1b_ref, r2w_ref, r2b_ref,
               o1_ref, o2_ref, acc_ref, *, nk, nc):
    pid = pl.program_id(0)

    @pl.when(pid == 0)
    def _():
        acc_ref[...] = jnp.zeros_like(acc_ref)

    b = bat_ref[...]
    g = lax.broadcasted_iota(jnp.int32, (nk, _G), 1)
    oh = (b == g).astype(jnp.float32)
    hx = jnp.concatenate([h_ref[..., :_F], jnp.ones((nk, 8), jnp.float32)],
                         axis=1)
    acc_ref[...] += lax.dot_general(
        oh, hx, (((0,), (0,)), ((), ())), preferred_element_type=jnp.float32)

    @pl.when(pid == nc - 1)
    def _():
        acc = acc_ref[...]
        mean = acc[:, :_F] / jnp.maximum(acc[:, _F:_F + 1], 1.0)
        o1_ref[...] = jnp.dot(mean, r1w_ref[...],
                              preferred_element_type=jnp.float32) + r1b_ref[...]
        o2_ref[...] = jnp.dot(mean, r2w_ref[...],
                              preferred_element_type=jnp.float32) + r2b_ref[...]


def _pool(h, batch2, r1_w, r1_b, r2_w, r2_b, nk):
    n = h.shape[0]
    nc = n // nk
    return pl.pallas_call(
        functools.partial(_pool_body, nk=nk, nc=nc),
        grid=(nc,),
        in_specs=[pl.BlockSpec((nk, _FP), lambda i: (i, 0)),
                  pl.BlockSpec((nk, 1), lambda i: (i, 0)),
                  pl.BlockSpec((_F, 1), lambda i: (0, 0)),
                  pl.BlockSpec((1, 1), lambda i: (0, 0)),
                  pl.BlockSpec((_F, 1), lambda i: (0, 0)),
                  pl.BlockSpec((1, 1), lambda i: (0, 0))],
        out_specs=(pl.BlockSpec((_G, 1), lambda i: (0, 0)),
                   pl.BlockSpec((_G, 1), lambda i: (0, 0))),
        out_shape=(jax.ShapeDtypeStruct((_G, 1), jnp.float32),
                   jax.ShapeDtypeStruct((_G, 1), jnp.float32)),
        scratch_shapes=[pltpu.VMEM((_G, _F + 8), jnp.float32)],
    )(h, batch2, r1_w, r1_b.reshape(1, 1), r2_w, r2_b.reshape(1, 1))


def _pad_to_grid(idx, total):
    e = idx.shape[0]
    padded = jnp.concatenate(
        [idx, jnp.zeros((total - e,), jnp.int32)]) if total > e else idx
    return padded.reshape(_W, total // (_W * _S), _S)


def kernel(pos, edge_index, batch,
           c1_w1, c1_b1, c1_w2, c1_b2,
           c2_w1, c2_b1, c2_w2, c2_b2,
           r1_w, r1_b, r2_w, r2_b):
    n = pos.shape[0]
    e = edge_index.shape[1]
    blk = _W * _S
    ep = ((e + blk - 1) // blk) * blk
    ek = 3200 if (e // 2) % 3200 == 0 else 2000
    nk = 2000 if n % 2000 == 0 else 200
    cntp = ((n + 16 + 12800 - 1) // 12800) * 12800

    src, dst = edge_index[0], edge_index[1]

    ends = edge_index.reshape(-1)
    e2 = ends.shape[0]
    e2p = ((e2 + blk - 1) // blk) * blk
    dump = n + (jnp.arange(e2p - e2, dtype=jnp.int32) % 16)
    ends_p = jnp.concatenate([ends, dump]).reshape(_W, e2p // blk, _S)
    cnt_ends = _sc_count(ends_p, cntp)
    mask = cnt_ends[:n] > 0
    assoc = jnp.cumsum(mask.astype(jnp.int32)).astype(jnp.int32) - 1

    d_sraw, s_sraw = lax.sort_key_val(dst, src)
    assoc_p = jnp.pad(assoc, (0, (-n) % _S))
    d_g, s_g = _sc_gather_e2(assoc_p, _pad_to_grid(d_sraw, ep),
                             _pad_to_grid(s_sraw, ep))
    d_sorted = d_g[:e]
    s_sorted = s_g[:e]

    dumpd = n + (jnp.arange(ep - e, dtype=jnp.int32) % 16)
    d2_p = jnp.concatenate([d_sorted, dumpd]).reshape(_W, ep // blk, _S)
    deg = _sc_count(d2_p, cntp)[:n]
    cum_end = jnp.cumsum(deg).astype(jnp.int32)
    has = (deg > 0)[:, None]
    last_pos = jnp.maximum(cum_end - 1, 0)

    half_e = e // 2
    hp = ((half_e + blk - 1) // blk) * blk
    sg_h = [_pad_to_grid(s_sorted[:half_e], hp),
            _pad_to_grid(s_sorted[half_e:], hp)]
    dg_pad = _pad_to_grid(d_sorted, ep)
    d_scan = jnp.concatenate(
        [d_sorted, jnp.full((ep - e,), n, jnp.int32)])[:, None]
    np2 = ((n + 2 * blk - 1) // (2 * blk)) * (2 * blk)
    lp1 = _pad_to_grid(jnp.minimum(last_pos, half_e - 1), np2)
    lp2 = _pad_to_grid(jnp.clip(last_pos - half_e, 0, half_e - 1), np2)
    in2 = last_pos[:, None] >= half_e

    def padc(w):
        return jnp.pad(w, ((0, 0), (0, _FP - w.shape[1])))

    def padrc(w):
        return jnp.pad(w, ((0, _FP - w.shape[0]), (0, _FP - w.shape[1])))

    def padv(v):
        return jnp.pad(v, (0, _FP - v.shape[0]))

    b_tab1 = _bprep(pos, padc(c1_w1[3:6]), nk)
    b_tab2 = _bprep(pos, padc(c2_w1[_F:_F + 3]), nk)
    xb_l = [_sc_gather1(b_tab1, dg_pad), _sc_gather1(b_tab2, dg_pad)]

    cd0 = jnp.full((1, 1), -1, jnp.int32)
    cm0 = jnp.zeros((8, _F), jnp.float32)
    lanes_ok = (jnp.arange(_FP, dtype=jnp.int32) < _F)[None, :]

    h = pos
    for layer in range(3):
        if layer == 0:
            wh, b1, w2, b2, b_tab = (padc(c1_w1[0:3]), padv(c1_b1),
                                     c1_w2, padv(c1_b2), b_tab1)
        else:
            wh, b1, w2, b2, b_tab = (padrc(c2_w1[0:_F]), padv(c2_b1),
                                     c2_w2, padv(c2_b2), b_tab2)
        xb = xb_l[min(layer, 1)]
        a_tab = _aprep(h, b_tab, wh, b1, nk)
        xa1 = _sc_gather1(a_tab, sg_h[0])
        xa2 = _sc_gather1(a_tab, sg_h[1])
        ms1, cd1, cm1 = _edge_scan(d_scan, xa1, xb, w2, cd0, cm0,
                                   ek, 0, half_e)
        ms2, _, _ = _edge_scan(d_scan, xa2, xb, w2, cd1, cm1,
                               ek, 1, half_e)
        hm1 = _sc_gather1(ms1, lp1)[:n]
        hm2 = _sc_gather1(ms2, lp2)[:n]
        hmax = jnp.where(in2, hm2, hm1)
        h = jnp.where(has & lanes_ok,
                      jnp.maximum(hmax + b2[None, :], 0.0), 0.0)

    return _pool(h, batch[:, None], r1_w, r1_b, r2_w, r2_b, nk)


# speedup vs baseline: 1.3930x; 1.3930x over previous
import functools

import jax
import jax.numpy as jnp
from jax import lax
from jax.experimental import pallas as pl
from jax.experimental.pallas import tpu as pltpu
from jax.experimental.pallas import tpu_sc as plsc

_F = 64
_FP = 128
_G = 64
_W = 32
_S = 128


def _mesh():
    return plsc.VectorSubcoreMesh(core_axis_name="c", subcore_axis_name="s",
                                  num_cores=2, num_subcores=16)


def _count_body(idx_hbm, out_hbm, idxv, onesv, zv, cnt_sh, *, steps, cntp):
    c = lax.axis_index("c")
    s = lax.axis_index("s")
    wid = c * 16 + s
    pltpu.sync_copy(idx_hbm.at[wid], idxv)
    for k in range(8):
        onesv[pl.ds(16 * k, 16)] = jnp.ones((16,), jnp.int32)

    nz = zv.shape[0]

    @pl.when(s == 0)
    def _():
        def zstep(i, _):
            zv[pl.ds(i * 16, 16)] = jnp.zeros((16,), jnp.int32)
            return 0
        lax.fori_loop(0, nz // 16, zstep, 0)
        for q in range(cntp // nz):
            pltpu.sync_copy(zv, cnt_sh.at[pl.ds(q * nz, nz)])

    plsc.subcore_barrier()

    def step(j, _):
        pltpu.sync_copy(onesv, cnt_sh.at[idxv.at[j]], add=True)
        return 0
    lax.fori_loop(0, steps, step, 0)

    plsc.subcore_barrier()
    sl = cntp // 16
    pltpu.sync_copy(cnt_sh.at[pl.ds(s * sl, sl)],
                    out_hbm.at[pl.ds(c * cntp + s * sl, sl)])


def _sc_count(idx3, cntp):
    steps = idx3.shape[1]
    out = pl.kernel(
        functools.partial(_count_body, steps=steps, cntp=cntp),
        out_type=jax.ShapeDtypeStruct((2 * cntp,), jnp.int32),
        mesh=_mesh(),
        scratch_types=[
            pltpu.VMEM((steps, _S), jnp.int32),
            pltpu.VMEM((_S,), jnp.int32),
            pltpu.VMEM((12800,), jnp.int32),
            pltpu.VMEM_SHARED((cntp,), jnp.int32),
        ],
    )(idx3)
    out = out.reshape(2, cntp)
    return out[0] + out[1]


def _gather2_body(taba, tabb, idxa, idxb, outa, outb,
                  iva, ivb, ra, rb, sa, sb, *, steps):
    c = lax.axis_index("c")
    s = lax.axis_index("s")
    wid = c * 16 + s
    pltpu.sync_copy(idxa.at[wid], iva)
    pltpu.sync_copy(idxb.at[wid], ivb)

    def step(j, _):
        ca = pltpu.async_copy(taba.at[iva.at[j]], ra, sa)
        cb = pltpu.async_copy(tabb.at[ivb.at[j]], rb, sb)
        ca.wait()
        cb.wait()
        pltpu.sync_copy(ra, outa.at[wid, j])
        pltpu.sync_copy(rb, outb.at[wid, j])
        return 0
    lax.fori_loop(0, steps, step, 0)


def _sc_gather2(taba, tabb, idxa3, idxb3):
    steps = idxa3.shape[1]
    sh = jax.ShapeDtypeStruct((_W, steps, _S, _FP), jnp.float32)
    outa, outb = pl.kernel(
        functools.partial(_gather2_body, steps=steps),
        out_type=(sh, sh),
        mesh=_mesh(),
        scratch_types=[
            pltpu.VMEM((steps, _S), jnp.int32),
            pltpu.VMEM((steps, _S), jnp.int32),
            pltpu.VMEM((_S, _FP), jnp.float32),
            pltpu.VMEM((_S, _FP), jnp.float32),
            pltpu.SemaphoreType.DMA,
            pltpu.SemaphoreType.DMA,
        ],
    )(taba, tabb, idxa3, idxb3)
    return (outa.reshape(_W * steps * _S, _FP),
            outb.reshape(_W * steps * _S, _FP))


def _gatherE2_body(taba, tabb, idxa, idxb, outa, outb,
                   iva, ivb, ra, rb, sa, sb, *, steps):
    c = lax.axis_index("c")
    s = lax.axis_index("s")
    wid = c * 16 + s
    pltpu.sync_copy(idxa.at[wid], iva)
    pltpu.sync_copy(idxb.at[wid], ivb)

    def step(j, _):
        ca = pltpu.async_copy(taba.at[iva.at[j]], ra, sa)
        cb = pltpu.async_copy(tabb.at[ivb.at[j]], rb, sb)
        ca.wait()
        cb.wait()
        pltpu.sync_copy(ra, outa.at[wid, j])
        pltpu.sync_copy(rb, outb.at[wid, j])
        return 0
    lax.fori_loop(0, steps, step, 0)


def _sc_gather_e2(tab, idxa3, idxb3):
    steps = idxa3.shape[1]
    sh = jax.ShapeDtypeStruct((_W, steps, _S), jnp.int32)
    outa, outb = pl.kernel(
        functools.partial(_gatherE2_body, steps=steps),
        out_type=(sh, sh),
        mesh=_mesh(),
        scratch_types=[
            pltpu.VMEM((steps, _S), jnp.int32),
            pltpu.VMEM((steps, _S), jnp.int32),
            pltpu.VMEM((_S,), jnp.int32),
            pltpu.VMEM((_S,), jnp.int32),
            pltpu.SemaphoreType.DMA,
            pltpu.SemaphoreType.DMA,
        ],
    )(tab, tab, idxa3, idxb3)
    return outa.reshape(-1), outb.reshape(-1)


def _gather1_body(tab, idx, out, iv, r0, r1, s0, s1, *, steps):
    c = lax.axis_index("c")
    s = lax.axis_index("s")
    wid = c * 16 + s
    pltpu.sync_copy(idx.at[wid], iv)

    def pair(jo, _):
        j0 = 2 * jo
        c0 = pltpu.async_copy(tab.at[iv.at[j0]], r0, s0)
        c1 = pltpu.async_copy(tab.at[iv.at[j0 + 1]], r1, s1)
        c0.wait()
        pltpu.sync_copy(r0, out.at[wid, j0])
        c1.wait()
        pltpu.sync_copy(r1, out.at[wid, j0 + 1])
        return 0
    lax.fori_loop(0, steps // 2, pair, 0)


def _sc_gather1(tab, idx3):
    steps = idx3.shape[1]
    out = pl.kernel(
        functools.partial(_gather1_body, steps=steps),
        out_type=jax.ShapeDtypeStruct((_W, steps, _S, _FP), jnp.float32),
        mesh=_mesh(),
        scratch_types=[
            pltpu.VMEM((steps, _S), jnp.int32),
            pltpu.VMEM((_S, _FP), jnp.float32),
            pltpu.VMEM((_S, _FP), jnp.float32),
            pltpu.SemaphoreType.DMA,
            pltpu.SemaphoreType.DMA,
        ],
    )(tab, idx3)
    return out.reshape(_W * steps * _S, _FP)


def _bprep_body(pos_ref, wp_ref, b_ref):
    b_ref[...] = jnp.dot(pos_ref[...], wp_ref[...],
                         preferred_element_type=jnp.float32)


def _aprep_body(h_ref, b_ref, wh_ref, b1_ref, a_ref):
    a_ref[...] = (jnp.dot(h_ref[...], wh_ref[...],
                          preferred_element_type=jnp.float32)
                  + b_ref[...] + b1_ref[...])


def _bprep(pos, wp, nk):
    n = pos.shape[0]
    return pl.pallas_call(
        _bprep_body,
        grid=(n // nk,),
        in_specs=[pl.BlockSpec((nk, 3), lambda i: (i, 0)),
                  pl.BlockSpec((3, _FP), lambda i: (0, 0))],
        out_specs=pl.BlockSpec((nk, _FP), lambda i: (i, 0)),
        out_shape=jax.ShapeDtypeStruct((n, _FP), jnp.float32),
    )(pos, wp)


def _aprep(h, b, wh, b1, nk):
    n, hw = h.shape
    return pl.pallas_call(
        _aprep_body,
        grid=(n // nk,),
        in_specs=[pl.BlockSpec((nk, hw), lambda i: (i, 0)),
                  pl.BlockSpec((nk, _FP), lambda i: (i, 0)),
                  pl.BlockSpec((hw, _FP), lambda i: (0, 0)),
                  pl.BlockSpec((1, _FP), lambda i: (0, 0))],
        out_specs=pl.BlockSpec((nk, _FP), lambda i: (i, 0)),
        out_shape=jax.ShapeDtypeStruct((n, _FP), jnp.float32),
    )(h, b, wh, b1.reshape(1, _FP))


def _edge_body(dst_ref, xa_ref, xb_ref, w2_ref, cdi_ref, cmi_ref,
               out_ref, cdo_ref, cmo_ref, cdst_ref, cmax_ref, *, ek, nc):
    pid = pl.program_id(0)
    x = jnp.maximum(xa_ref[:, :_F] - xb_ref[:, :_F], 0.0)
    m = jnp.dot(x, w2_ref[...], preferred_element_type=jnp.float32)
    d = dst_ref[...]
    rows = lax.broadcasted_iota(jnp.int32, (ek, 1), 0)
    s = 1
    while s < ek:
        d_s = jnp.where(rows >= s, pltpu.roll(d, s, 0), -1)
        m_s = pltpu.roll(m, s, 0)
        take = d_s == d
        m = jnp.where(take, jnp.maximum(m, m_s), m)
        s *= 2

    @pl.when(pid == 0)
    def _():
        cdst_ref[...] = cdi_ref[...]
        cmax_ref[...] = cmi_ref[...]

    m = jnp.where(d == cdst_ref[...], jnp.maximum(m, cmax_ref[0:1, :]), m)
    out_ref[...] = jnp.concatenate(
        [m, jnp.zeros((ek, _FP - _F), jnp.float32)], axis=1)
    cdst_ref[...] = d[ek - 1:ek, 0:1]
    cmax_ref[0:1, :] = m[ek - 1:ek, :]

    @pl.when(pid == nc - 1)
    def _():
        cdo_ref[...] = cdst_ref[...]
        cmo_ref[...] = cmax_ref[...]


def _edge_scan(dst2, xa, xb, w2, cdi, cmi, ek, half, nedges):
    nc = nedges // ek
    off = half * nc
    return pl.pallas_call(
        functools.partial(_edge_body, ek=ek, nc=nc),
        grid=(nc,),
        in_specs=[pl.BlockSpec((ek, 1), lambda i: (i + off, 0)),
                  pl.BlockSpec((ek, _FP), lambda i: (i, 0)),
                  pl.BlockSpec((ek, _FP), lambda i: (i + off, 0)),
                  pl.BlockSpec((_F, _F), lambda i: (0, 0)),
                  pl.BlockSpec((1, 1), lambda i: (0, 0)),
                  pl.BlockSpec((8, _F), lambda i: (0, 0))],
        out_specs=(pl.BlockSpec((ek, _FP), lambda i: (i, 0)),
                   pl.BlockSpec((1, 1), lambda i: (0, 0)),
                   pl.BlockSpec((8, _F), lambda i: (0, 0))),
        out_shape=(jax.ShapeDtypeStruct((nedges, _FP), jnp.float32),
                   jax.ShapeDtypeStruct((1, 1), jnp.int32),
                   jax.ShapeDtypeStruct((8, _F), jnp.float32)),
        scratch_shapes=[pltpu.VMEM((1, 1), jnp.int32),
                        pltpu.VMEM((8, _F), jnp.float32)],
    )(dst2, xa, xb, w2, cdi, cmi)


def _pool_body(h_ref, bat_ref, r1w_ref, r1b_ref, r2w_ref, r2b_ref,
               o1_ref, o2_ref, acc_ref, *, nk, nc):
    pid = pl.program_id(0)

    @pl.when(pid == 0)
    def _():
        acc_ref[...] = jnp.zeros_like(acc_ref)

    b = bat_ref[...]
    g = lax.broadcasted_iota(jnp.int32, (nk, _G), 1)
    oh = (b == g).astype(jnp.float32)
    hx = jnp.concatenate([h_ref[..., :_F], jnp.ones((nk, 8), jnp.float32)],
                         axis=1)
    acc_ref[...] += lax.dot_general(
        oh, hx, (((0,), (0,)), ((), ())), preferred_element_type=jnp.float32)

    @pl.when(pid == nc - 1)
    def _():
        acc = acc_ref[...]
        mean = acc[:, :_F] / jnp.maximum(acc[:, _F:_F + 1], 1.0)
        o1_ref[...] = jnp.dot(mean, r1w_ref[...],
                              preferred_element_type=jnp.float32) + r1b_ref[...]
        o2_ref[...] = jnp.dot(mean, r2w_ref[...],
                              preferred_element_type=jnp.float32) + r2b_ref[...]


def _pool(h, batch2, r1_w, r1_b, r2_w, r2_b, nk):
    n = h.shape[0]
    nc = n // nk
    return pl.pallas_call(
        functools.partial(_pool_body, nk=nk, nc=nc),
        grid=(nc,),
        in_specs=[pl.BlockSpec((nk, _FP), lambda i: (i, 0)),
                  pl.BlockSpec((nk, 1), lambda i: (i, 0)),
                  pl.BlockSpec((_F, 1), lambda i: (0, 0)),
                  pl.BlockSpec((1, 1), lambda i: (0, 0)),
                  pl.BlockSpec((_F, 1), lambda i: (0, 0)),
                  pl.BlockSpec((1, 1), lambda i: (0, 0))],
        out_specs=(pl.BlockSpec((_G, 1), lambda i: (0, 0)),
                   pl.BlockSpec((_G, 1), lambda i: (0, 0))),
        out_shape=(jax.ShapeDtypeStruct((_G, 1), jnp.float32),
                   jax.ShapeDtypeStruct((_G, 1), jnp.float32)),
        scratch_shapes=[pltpu.VMEM((_G, _F + 8), jnp.float32)],
    )(h, batch2, r1_w, r1_b.reshape(1, 1), r2_w, r2_b.reshape(1, 1))


def _pad_to_grid(idx, total):
    e = idx.shape[0]
    padded = jnp.concatenate(
        [idx, jnp.zeros((total - e,), jnp.int32)]) if total > e else idx
    return padded.reshape(_W, total // (_W * _S), _S)


def kernel(pos, edge_index, batch,
           c1_w1, c1_b1, c1_w2, c1_b2,
           c2_w1, c2_b1, c2_w2, c2_b2,
           r1_w, r1_b, r2_w, r2_b):
    n = pos.shape[0]
    e = edge_index.shape[1]
    blk = _W * _S
    ep = ((e + blk - 1) // blk) * blk
    ek = 3200 if (e // 2) % 3200 == 0 else 2000
    nk = 2000 if n % 2000 == 0 else 200
    cntp = ((n + 16 + 12800 - 1) // 12800) * 12800

    src, dst = edge_index[0], edge_index[1]

    ends = edge_index.reshape(-1)
    e2 = ends.shape[0]
    e2p = ((e2 + blk - 1) // blk) * blk
    dump = n + (jnp.arange(e2p - e2, dtype=jnp.int32) % 16)
    ends_p = jnp.concatenate([ends, dump]).reshape(_W, e2p // blk, _S)
    cnt_ends = _sc_count(ends_p, cntp)
    mask = cnt_ends[:n] > 0
    assoc = jnp.cumsum(mask.astype(jnp.int32)).astype(jnp.int32) - 1

    d_sraw, s_sraw = lax.sort_key_val(dst, src)
    assoc_p = jnp.pad(assoc, (0, (-n) % _S))
    d_g, s_g = _sc_gather_e2(assoc_p, _pad_to_grid(d_sraw, ep),
                             _pad_to_grid(s_sraw, ep))
    d_sorted = d_g[:e]
    s_sorted = s_g[:e]

    dumpd = n + (jnp.arange(ep - e, dtype=jnp.int32) % 16)
    d2_p = jnp.concatenate([d_sorted, dumpd]).reshape(_W, ep // blk, _S)
    deg = _sc_count(d2_p, cntp)[:n]
    cum_end = jnp.cumsum(deg).astype(jnp.int32)
    has = (deg > 0)[:, None]
    last_pos = jnp.maximum(cum_end - 1, 0)

    sg_pad = _pad_to_grid(s_sorted, ep)
    dg_pad = _pad_to_grid(d_sorted, ep)
    d_scan = jnp.concatenate(
        [d_sorted, jnp.full((ep - e,), n, jnp.int32)])[:, None]
    np2 = ((n + 2 * blk - 1) // (2 * blk)) * (2 * blk)
    lp_pad = _pad_to_grid(last_pos, np2)

    def padc(w):
        return jnp.pad(w, ((0, 0), (0, _FP - w.shape[1])))

    def padrc(w):
        return jnp.pad(w, ((0, _FP - w.shape[0]), (0, _FP - w.shape[1])))

    def padv(v):
        return jnp.pad(v, (0, _FP - v.shape[0]))

    b_tab1 = _bprep(pos, padc(c1_w1[3:6]), nk)
    b_tab2 = _bprep(pos, padc(c2_w1[_F:_F + 3]), nk)
    xb_l = [_sc_gather1(b_tab1, dg_pad), _sc_gather1(b_tab2, dg_pad)]

    cd0 = jnp.full((1, 1), -1, jnp.int32)
    cm0 = jnp.zeros((8, _F), jnp.float32)
    lanes_ok = (jnp.arange(_FP, dtype=jnp.int32) < _F)[None, :]

    h = pos
    for layer in range(3):
        if layer == 0:
            wh, b1, w2, b2, b_tab = (padc(c1_w1[0:3]), padv(c1_b1),
                                     c1_w2, padv(c1_b2), b_tab1)
        else:
            wh, b1, w2, b2, b_tab = (padrc(c2_w1[0:_F]), padv(c2_b1),
                                     c2_w2, padv(c2_b2), b_tab2)
        xb = xb_l[min(layer, 1)]
        a_tab = _aprep(h, b_tab, wh, b1, nk)
        xa = _sc_gather1(a_tab, sg_pad)
        mscan, _, _ = _edge_scan(d_scan, xa, xb, w2, cd0, cm0, ek, 0, e)
        hmax = _sc_gather1(mscan, lp_pad)[:n]
        h = jnp.where(has & lanes_ok,
                      jnp.maximum(hmax + b2[None, :], 0.0), 0.0)

    return _pool(h, batch[:, None], r1_w, r1_b, r2_w, r2_b, nk)
